# baseline (speedup 1.0000x reference)
import functools

import jax
import jax.numpy as jnp
from jax import lax
from jax.experimental import pallas as pl
from jax.experimental.pallas import tpu as pltpu
from jax.experimental.pallas import tpu_sc as plsc

VOCAB = 1000000
EMBED_DIM = 64
BATCH = 16384
CTX = 20

NUM_CORES = 2
NUM_SUBCORES = 16
NUM_WORKERS = NUM_CORES * NUM_SUBCORES
B_PER_W = BATCH // NUM_WORKERS
CHUNK = 32
NCHUNK = B_PER_W // CHUNK
ROWS_PER_CHUNK = CHUNK * CTX
GATHER_SPLIT = 640
NGATHER = ROWS_PER_CHUNK // GATHER_SPLIT
IDX_PER_W = B_PER_W * CTX
LANES = 16
DSLICES = EMBED_DIM // LANES
NBUF = 2


def _cbow_body(idx_hbm, table_hbm, out_hbm, idx_v, rows_v, out_v, sems, osems):
    wid = lax.axis_index("s") * NUM_CORES + lax.axis_index("c")
    base_row = wid * B_PER_W
    scale = jnp.float32(1.0 / CTX)

    pltpu.sync_copy(idx_hbm.at[pl.ds(wid * IDX_PER_W, IDX_PER_W)], idx_v)

    def issue(ci, slot):
        for g in range(ROWS_PER_CHUNK // LANES):
            idx_vec = idx_v[pl.ds(ci * ROWS_PER_CHUNK + g * LANES, LANES)]
            pltpu.make_async_copy(
                table_hbm.at[idx_vec],
                rows_v.at[slot].at[pl.ds(g * LANES, LANES)],
                sems.at[slot],
            ).start()

    def drain(ci, slot):
        del ci
        pltpu.make_async_copy(
            table_hbm.at[pl.ds(0, ROWS_PER_CHUNK)],
            rows_v.at[slot],
            sems.at[slot],
        ).wait()

    def out_copy(ci, slot):
        return pltpu.make_async_copy(
            out_v.at[slot],
            out_hbm.at[pl.ds((base_row + ci * CHUNK) // 2, CHUNK // 2)],
            osems.at[slot],
        )

    def pool(ci, slot, first):
        def pool_body(b, inner):
            r0 = b * CTX
            accs = [
                rows_v[slot, r0, pl.ds(d * LANES, LANES)] for d in range(DSLICES)
            ]
            for c in range(1, CTX):
                for d in range(DSLICES):
                    accs[d] = accs[d] + rows_v[slot, r0 + c, pl.ds(d * LANES, LANES)]
            ob = b >> 1
            oh = (b & 1) * EMBED_DIM
            for d in range(DSLICES):
                out_v[slot, ob, pl.ds(oh + d * LANES, LANES)] = accs[d] * scale
            return inner

        @pl.when(jnp.logical_not(first))
        def _():
            out_copy(ci, slot).wait()

        lax.fori_loop(0, CHUNK, pool_body, 0)
        out_copy(ci, slot).start()

    issue(0, 0)

    def chunk_pair(cp, carry):
        ci0 = cp * NBUF
        for b in range(NBUF):
            ci = ci0 + b
            nxt = ci + 1

            nslot = (b + 1) % NBUF

            @pl.when(nxt < NCHUNK)
            def _():
                issue(nxt, nslot)

            drain(ci, b)
            pool(ci, b, ci < NBUF)
        return carry

    lax.fori_loop(0, NCHUNK // NBUF, chunk_pair, 0)

    for b in range(NBUF):
        out_copy(NCHUNK - NBUF + b, b).wait()


@jax.jit
def _cbow(idx_flat, table):
    mesh = plsc.VectorSubcoreMesh(core_axis_name="c", subcore_axis_name="s")
    kern = functools.partial(
        pl.kernel,
        mesh=mesh,
        out_type=jax.ShapeDtypeStruct((BATCH // 2, 2 * EMBED_DIM), jnp.float32),
        scratch_types=[
            pltpu.VMEM((IDX_PER_W,), jnp.int32),
            pltpu.VMEM((NBUF, ROWS_PER_CHUNK, EMBED_DIM), jnp.float32),
            pltpu.VMEM((NBUF, CHUNK // 2, 2 * EMBED_DIM), jnp.float32),
            pltpu.SemaphoreType.DMA((NBUF,)),
            pltpu.SemaphoreType.DMA((NBUF,)),
        ],
        compiler_params=pltpu.CompilerParams(use_tc_tiling_on_sc=False),
    )(_cbow_body)
    return kern(idx_flat, table)


def kernel(inputs, table):
    idx_flat = inputs.astype(jnp.int32).reshape(-1)
    return _cbow(idx_flat, table).reshape(BATCH, EMBED_DIM)


# speedup vs baseline: 1.0017x; 1.0017x over previous
import functools

import jax
import jax.numpy as jnp
from jax import lax
from jax.experimental import pallas as pl
from jax.experimental.pallas import tpu as pltpu
from jax.experimental.pallas import tpu_sc as plsc

VOCAB = 1000000
EMBED_DIM = 64
BATCH = 16384
CTX = 20

NUM_CORES = 2
NUM_SUBCORES = 16
NUM_WORKERS = NUM_CORES * NUM_SUBCORES
B_PER_W = BATCH // NUM_WORKERS
CHUNK = 32
NCHUNK = B_PER_W // CHUNK
ROWS_PER_CHUNK = CHUNK * CTX
GATHER_SPLIT = 640
NGATHER = ROWS_PER_CHUNK // GATHER_SPLIT
IDX_PER_W = B_PER_W * CTX
LANES = 16
DSLICES = EMBED_DIM // LANES
NBUF = 2


def _cbow_body(idx_hbm, table_hbm, out_hbm, idx_v, rows_v, out_v, sems, osems):
    wid = lax.axis_index("s") * NUM_CORES + lax.axis_index("c")
    base_row = wid * B_PER_W
    scale = jnp.float32(1.0 / CTX)

    pltpu.sync_copy(
        idx_hbm.at[pl.ds(wid * IDX_PER_W, ROWS_PER_CHUNK)],
        idx_v.at[pl.ds(0, ROWS_PER_CHUNK)],
    )

    def issue(ci, slot):
        for g in range(ROWS_PER_CHUNK // LANES):
            idx_vec = idx_v[pl.ds(ci * ROWS_PER_CHUNK + g * LANES, LANES)]
            pltpu.make_async_copy(
                table_hbm.at[idx_vec],
                rows_v.at[slot].at[pl.ds(g * LANES, LANES)],
                sems.at[slot],
            ).start()

    def drain(ci, slot):
        del ci
        pltpu.make_async_copy(
            table_hbm.at[pl.ds(0, ROWS_PER_CHUNK)],
            rows_v.at[slot],
            sems.at[slot],
        ).wait()

    def out_copy(ci, slot):
        return pltpu.make_async_copy(
            out_v.at[slot],
            out_hbm.at[pl.ds((base_row + ci * CHUNK) // 2, CHUNK // 2)],
            osems.at[slot],
        )

    def pool(ci, slot, first):
        def pool_body(b, inner):
            r0 = b * CTX
            accs = [
                rows_v[slot, r0, pl.ds(d * LANES, LANES)] for d in range(DSLICES)
            ]
            for c in range(1, CTX):
                for d in range(DSLICES):
                    accs[d] = accs[d] + rows_v[slot, r0 + c, pl.ds(d * LANES, LANES)]
            ob = b >> 1
            oh = (b & 1) * EMBED_DIM
            for d in range(DSLICES):
                out_v[slot, ob, pl.ds(oh + d * LANES, LANES)] = accs[d] * scale
            return inner

        @pl.when(jnp.logical_not(first))
        def _():
            out_copy(ci, slot).wait()

        lax.fori_loop(0, CHUNK, pool_body, 0)
        out_copy(ci, slot).start()

    issue(0, 0)
    pltpu.sync_copy(
        idx_hbm.at[pl.ds(wid * IDX_PER_W + ROWS_PER_CHUNK,
                         IDX_PER_W - ROWS_PER_CHUNK)],
        idx_v.at[pl.ds(ROWS_PER_CHUNK, IDX_PER_W - ROWS_PER_CHUNK)],
    )

    def chunk_pair(cp, carry):
        ci0 = cp * NBUF
        for b in range(NBUF):
            ci = ci0 + b
            nxt = ci + 1

            nslot = (b + 1) % NBUF

            @pl.when(nxt < NCHUNK)
            def _():
                issue(nxt, nslot)

            drain(ci, b)
            pool(ci, b, ci < NBUF)
        return carry

    lax.fori_loop(0, NCHUNK // NBUF, chunk_pair, 0)

    for b in range(NBUF):
        out_copy(NCHUNK - NBUF + b, b).wait()


@jax.jit
def _cbow(idx_flat, table):
    mesh = plsc.VectorSubcoreMesh(core_axis_name="c", subcore_axis_name="s")
    kern = functools.partial(
        pl.kernel,
        mesh=mesh,
        out_type=jax.ShapeDtypeStruct((BATCH // 2, 2 * EMBED_DIM), jnp.float32),
        scratch_types=[
            pltpu.VMEM((IDX_PER_W,), jnp.int32),
            pltpu.VMEM((NBUF, ROWS_PER_CHUNK, EMBED_DIM), jnp.float32),
            pltpu.VMEM((NBUF, CHUNK // 2, 2 * EMBED_DIM), jnp.float32),
            pltpu.SemaphoreType.DMA((NBUF,)),
            pltpu.SemaphoreType.DMA((NBUF,)),
        ],
        compiler_params=pltpu.CompilerParams(use_tc_tiling_on_sc=False),
    )(_cbow_body)
    return kern(idx_flat, table)


def kernel(inputs, table):
    idx_flat = inputs.astype(jnp.int32).reshape(-1)
    return _cbow(idx_flat, table).reshape(BATCH, EMBED_DIM)


# speedup vs baseline: 1.0018x; 1.0001x over previous
import functools

import jax
import jax.numpy as jnp
from jax import lax
from jax.experimental import pallas as pl
from jax.experimental.pallas import tpu as pltpu
from jax.experimental.pallas import tpu_sc as plsc

VOCAB = 1000000
EMBED_DIM = 64
BATCH = 16384
CTX = 20

NUM_CORES = 2
NUM_SUBCORES = 16
NUM_WORKERS = NUM_CORES * NUM_SUBCORES
B_PER_W = BATCH // NUM_WORKERS
CHUNK = 32
NCHUNK = B_PER_W // CHUNK
ROWS_PER_CHUNK = CHUNK * CTX
IDX_PER_W = B_PER_W * CTX
LANES = 16
DSLICES = EMBED_DIM // LANES
NBUF = 2


def _cbow_body(idx_hbm, table_hbm, out_hbm, idx_v, rows_v, out_v, sems, osems):
    wid = lax.axis_index("s") * NUM_CORES + lax.axis_index("c")
    base_row = wid * B_PER_W
    scale = jnp.float32(1.0 / CTX)

    pltpu.sync_copy(
        idx_hbm.at[pl.ds(wid * IDX_PER_W, ROWS_PER_CHUNK)],
        idx_v.at[pl.ds(0, ROWS_PER_CHUNK)],
    )

    def issue(ci, slot):
        for g in range(ROWS_PER_CHUNK // LANES):
            idx_vec = idx_v[pl.ds(ci * ROWS_PER_CHUNK + g * LANES, LANES)]
            pltpu.make_async_copy(
                table_hbm.at[idx_vec],
                rows_v.at[slot].at[pl.ds(g * LANES, LANES)],
                sems.at[slot],
            ).start()

    def drain(ci, slot):
        del ci
        pltpu.make_async_copy(
            table_hbm.at[pl.ds(0, ROWS_PER_CHUNK)],
            rows_v.at[slot],
            sems.at[slot],
        ).wait()

    def out_copy(ci, slot):
        return pltpu.make_async_copy(
            out_v.at[slot],
            out_hbm.at[pl.ds((base_row + ci * CHUNK) // 2, CHUNK // 2)],
            osems.at[slot],
        )

    def pool(ci, slot, first):
        def pool_body(b, inner):
            r0 = b * CTX
            accs = [
                rows_v[slot, r0, pl.ds(d * LANES, LANES)] for d in range(DSLICES)
            ]
            for c in range(1, CTX):
                for d in range(DSLICES):
                    accs[d] = accs[d] + rows_v[slot, r0 + c, pl.ds(d * LANES, LANES)]
            ob = b >> 1
            oh = (b & 1) * EMBED_DIM
            for d in range(DSLICES):
                out_v[slot, ob, pl.ds(oh + d * LANES, LANES)] = accs[d] * scale
            return inner

        @pl.when(jnp.logical_not(first))
        def _():
            out_copy(ci, slot).wait()

        lax.fori_loop(0, CHUNK, pool_body, 0)
        out_copy(ci, slot).start()

    issue(0, 0)
    pltpu.sync_copy(
        idx_hbm.at[pl.ds(wid * IDX_PER_W + ROWS_PER_CHUNK,
                         IDX_PER_W - ROWS_PER_CHUNK)],
        idx_v.at[pl.ds(ROWS_PER_CHUNK, IDX_PER_W - ROWS_PER_CHUNK)],
    )

    def chunk_pair(cp, carry):
        ci0 = cp * NBUF
        for b in range(NBUF):
            ci = ci0 + b
            nxt = ci + 1

            nslot = (b + 1) % NBUF

            @pl.when(nxt < NCHUNK)
            def _():
                issue(nxt, nslot)

            drain(ci, b)
            pool(ci, b, ci < NBUF)
        return carry

    lax.fori_loop(0, NCHUNK // NBUF, chunk_pair, 0)

    for b in range(NBUF):
        out_copy(NCHUNK - NBUF + b, b).wait()


@jax.jit
def _cbow(idx_flat, table):
    mesh = plsc.VectorSubcoreMesh(core_axis_name="c", subcore_axis_name="s")
    kern = functools.partial(
        pl.kernel,
        mesh=mesh,
        out_type=jax.ShapeDtypeStruct((BATCH // 2, 2 * EMBED_DIM), jnp.float32),
        scratch_types=[
            pltpu.VMEM((IDX_PER_W,), jnp.int32),
            pltpu.VMEM((NBUF, ROWS_PER_CHUNK, EMBED_DIM), jnp.float32),
            pltpu.VMEM((NBUF, CHUNK // 2, 2 * EMBED_DIM), jnp.float32),
            pltpu.SemaphoreType.DMA((NBUF,)),
            pltpu.SemaphoreType.DMA((NBUF,)),
        ],
        compiler_params=pltpu.CompilerParams(use_tc_tiling_on_sc=False),
    )(_cbow_body)
    return kern(idx_flat, table)


def kernel(inputs, table):
    idx_flat = inputs.astype(jnp.int32).reshape(-1)
    return _cbow(idx_flat, table).reshape(BATCH, EMBED_DIM)
